# baseline (speedup 1.0000x reference)
import functools

import jax
import jax.numpy as jnp
import numpy as np
from jax import lax
from jax.experimental import pallas as pl
from jax.experimental.pallas import tpu as pltpu
from jax.experimental.pallas import tpu_sc as plsc

B, N = 1024, 256
SIZES = list(range(5, 13))
OUT_K = 12 * 12 * 8
NROW = 43
NROW_PAD = 48
NVEC = OUT_K // 16
NBUF = 8
TBL_SROWS = 0
TBL_EROW = NROW_PAD
TBL_ECOL = TBL_EROW + OUT_K
TBL_MASK = TBL_ECOL + OUT_K
TBL_LEN = TBL_MASK + OUT_K


def _build_tables():
    per = {s: np.round(np.linspace(0.0, float(N - 1), s).astype(np.float64)).astype(np.int32)
           for s in SIZES}
    union = sorted({int(v) for s in SIZES for v in per[s]})
    pos = {v: i for i, v in enumerate(union)}
    srows = np.array(union + [union[-1]] * (NROW_PAD - len(union)), np.int32)
    erow = np.zeros(OUT_K, np.int32)
    ecol = np.zeros(OUT_K, np.int32)
    mask = np.zeros(OUT_K, np.float32)
    for i in range(12):
        for j in range(12):
            for si, s in enumerate(SIZES):
                k = i * 96 + j * 8 + si
                if i < s and j < s:
                    erow[k] = pos[int(per[s][i])]
                    ecol[k] = int(per[s][j])
                    mask[k] = 1.0
    return np.concatenate([srows, erow, ecol, mask.view(np.int32)])


_TBL = _build_tables()

_INFO = plsc.get_sparse_core_info()
_NC, _NS = _INFO.num_cores, _INFO.num_subcores
_NW = _NC * _NS
_B_PER_W = B // _NW

_MESH = plsc.VectorSubcoreMesh(core_axis_name="c", subcore_axis_name="s")

_SCRATCH = (
    [pltpu.VMEM((TBL_LEN,), jnp.int32)]
    + [pltpu.VMEM((NROW,), jnp.int32) for _ in range(NBUF)]
    + [pltpu.VMEM((NROW, N), jnp.float32) for _ in range(NBUF)]
    + [pltpu.VMEM((OUT_K,), jnp.float32) for _ in range(NBUF)]
    + [pltpu.SemaphoreType.DMA for _ in range(2 * NBUF)]
)


@functools.partial(
    pl.kernel,
    mesh=_MESH,
    compiler_params=pltpu.CompilerParams(needs_layout_passes=False),
    out_type=jax.ShapeDtypeStruct((B, OUT_K), jnp.float32),
    scratch_types=_SCRATCH,
)
def _sc_gather(table, tbl, out, *scr):
    tbl_v = scr[0]
    idx_v = scr[1:1 + NBUF]
    rows_v = scr[1 + NBUF:1 + 2 * NBUF]
    out_v = scr[1 + 2 * NBUF:1 + 3 * NBUF]
    gsem = scr[1 + 3 * NBUF:1 + 4 * NBUF]
    ssem = scr[1 + 4 * NBUF:1 + 5 * NBUF]

    wid = lax.axis_index("s") * _NC + lax.axis_index("c")
    base = wid * _B_PER_W
    pltpu.sync_copy(tbl, tbl_v)

    lanes2 = jnp.arange(32, 48, dtype=jnp.int32)
    tail_mask = jnp.arange(16, dtype=jnp.int32) < (NROW - 32)

    def start_gather(s, bb):
        for c in range(2):
            sl = pl.ds(c * 16, 16)
            idx_v[s][sl] = tbl_v[sl] + bb * N
        vals = tbl_v[pl.ds(32, 16)] + bb * N
        plsc.store_scatter(idx_v[s], [lanes2], vals, mask=tail_mask)
        pltpu.async_copy(table.at[idx_v[s]], rows_v[s], gsem[s])

    for s in range(NBUF):
        start_gather(s, base + s)

    def body(g, _):
        for s in range(NBUF):
            i = g * NBUF + s
            bb = base + i
            pltpu.make_async_copy(table.at[idx_v[s]], rows_v[s], gsem[s]).wait()

            @pl.when(i >= NBUF)
            def _():
                pltpu.make_async_copy(out_v[s], out.at[bb - NBUF], ssem[s]).wait()

            for k in range(NVEC):
                sl = pl.ds(k * 16, 16)
                gth = plsc.load_gather(
                    rows_v[s], [tbl_v[pl.ds(TBL_EROW + k * 16, 16)],
                                tbl_v[pl.ds(TBL_ECOL + k * 16, 16)]])
                msk = plsc.bitcast(tbl_v[pl.ds(TBL_MASK + k * 16, 16)], jnp.float32)
                out_v[s][sl] = gth * msk
            pltpu.async_copy(out_v[s], out.at[bb], ssem[s])

            @pl.when(i < _B_PER_W - NBUF)
            def _():
                start_gather(s, bb + NBUF)
        return ()

    lax.fori_loop(0, _B_PER_W // NBUF, body, ())

    for s in range(NBUF):
        pltpu.make_async_copy(
            out_v[s], out.at[base + _B_PER_W - NBUF + s], ssem[s]).wait()


def kernel(inputs):
    table = inputs.reshape(B * N, N)
    out2d = _sc_gather(table, jnp.asarray(_TBL))
    return out2d.reshape(B, 12, 12, 8)


# speedup vs baseline: 1.1915x; 1.1915x over previous
import functools

import jax
import jax.numpy as jnp
import numpy as np
from jax import lax
from jax.experimental import pallas as pl
from jax.experimental.pallas import tpu as pltpu
from jax.experimental.pallas import tpu_sc as plsc

B, N = 1024, 256
SIZES = list(range(5, 13))
OUT_K = 12 * 12 * 8
NROW = 43
NROW_PAD = 48
NVEC = OUT_K // 16
NBUF = 2
TBL_SROWS = 0
TBL_EROW = NROW_PAD
TBL_ECOL = TBL_EROW + OUT_K
TBL_MASK = TBL_ECOL + OUT_K
TBL_LEN = TBL_MASK + OUT_K


def _build_tables():
    per = {s: np.round(np.linspace(0.0, float(N - 1), s).astype(np.float64)).astype(np.int32)
           for s in SIZES}
    union = sorted({int(v) for s in SIZES for v in per[s]})
    pos = {v: i for i, v in enumerate(union)}
    srows = np.array(union + [union[-1]] * (NROW_PAD - len(union)), np.int32)
    erow = np.zeros(OUT_K, np.int32)
    ecol = np.zeros(OUT_K, np.int32)
    mask = np.zeros(OUT_K, np.float32)
    for i in range(12):
        for j in range(12):
            for si, s in enumerate(SIZES):
                k = i * 96 + j * 8 + si
                if i < s and j < s:
                    erow[k] = pos[int(per[s][i])]
                    ecol[k] = int(per[s][j])
                    mask[k] = 1.0
    return np.concatenate([srows, erow, ecol, mask.view(np.int32)])


_TBL = _build_tables()

_INFO = plsc.get_sparse_core_info()
_NC, _NS = _INFO.num_cores, _INFO.num_subcores
_NW = _NC * _NS
_B_PER_W = B // _NW

_MESH = plsc.VectorSubcoreMesh(core_axis_name="c", subcore_axis_name="s")

_SCRATCH = (
    [pltpu.VMEM((TBL_LEN,), jnp.int32)]
    + [pltpu.VMEM((NROW,), jnp.int32) for _ in range(NBUF)]
    + [pltpu.VMEM((NROW, N), jnp.float32) for _ in range(NBUF)]
    + [pltpu.VMEM((OUT_K,), jnp.float32) for _ in range(NBUF)]
    + [pltpu.SemaphoreType.DMA for _ in range(2 * NBUF)]
)


@functools.partial(
    pl.kernel,
    mesh=_MESH,
    compiler_params=pltpu.CompilerParams(needs_layout_passes=False),
    out_type=jax.ShapeDtypeStruct((B, OUT_K), jnp.float32),
    scratch_types=_SCRATCH,
)
def _sc_gather(table, tbl, out, *scr):
    tbl_v = scr[0]
    idx_v = scr[1:1 + NBUF]
    rows_v = scr[1 + NBUF:1 + 2 * NBUF]
    out_v = scr[1 + 2 * NBUF:1 + 3 * NBUF]
    gsem = scr[1 + 3 * NBUF:1 + 4 * NBUF]
    ssem = scr[1 + 4 * NBUF:1 + 5 * NBUF]

    wid = lax.axis_index("s") * _NC + lax.axis_index("c")
    base = wid * _B_PER_W
    pltpu.sync_copy(tbl, tbl_v)

    lanes2 = jnp.arange(32, 48, dtype=jnp.int32)
    tail_mask = jnp.arange(16, dtype=jnp.int32) < (NROW - 32)

    def start_gather(s, bb):
        for c in range(2):
            sl = pl.ds(c * 16, 16)
            idx_v[s][sl] = tbl_v[sl] + bb * N
        vals = tbl_v[pl.ds(32, 16)] + bb * N
        plsc.store_scatter(idx_v[s], [lanes2], vals, mask=tail_mask)
        pltpu.async_copy(table.at[idx_v[s]], rows_v[s], gsem[s])

    for s in range(NBUF):
        start_gather(s, base + s)

    def body(g, _):
        for s in range(NBUF):
            i = g * NBUF + s
            bb = base + i
            pltpu.make_async_copy(table.at[idx_v[s]], rows_v[s], gsem[s]).wait()

            @pl.when(i >= NBUF)
            def _():
                pltpu.make_async_copy(out_v[s], out.at[bb - NBUF], ssem[s]).wait()

            for k in range(NVEC):
                sl = pl.ds(k * 16, 16)
                gth = plsc.load_gather(
                    rows_v[s], [tbl_v[pl.ds(TBL_EROW + k * 16, 16)],
                                tbl_v[pl.ds(TBL_ECOL + k * 16, 16)]])
                msk = plsc.bitcast(tbl_v[pl.ds(TBL_MASK + k * 16, 16)], jnp.float32)
                out_v[s][sl] = gth * msk
            pltpu.async_copy(out_v[s], out.at[bb], ssem[s])

            @pl.when(i < _B_PER_W - NBUF)
            def _():
                start_gather(s, bb + NBUF)
        return ()

    lax.fori_loop(0, _B_PER_W // NBUF, body, ())

    for s in range(NBUF):
        pltpu.make_async_copy(
            out_v[s], out.at[base + _B_PER_W - NBUF + s], ssem[s]).wait()


def kernel(inputs):
    table = inputs.reshape(B * N, N)
    out2d = _sc_gather(table, jnp.asarray(_TBL))
    return out2d.reshape(B, 12, 12, 8)
